# Initial kernel scaffold; baseline (speedup 1.0000x reference)
import jax
import jax.numpy as jnp
from jax.experimental import pallas as pl


def kernel(input_matrix, adj, W1, b1, W2, b2):
    raise NotImplementedError("write your pallas kernel here")


# speedup vs baseline: 1.4528x; 1.4528x over previous
import jax
import jax.numpy as jnp
from jax.experimental import pallas as pl


def _pass1_body(adj_ref, x_ref, w1_ref, b1_ref, w2_ref, hw2_ref, deg_ref):
    a = adj_ref[...]
    deg = jnp.maximum(jnp.sum(a, axis=1, keepdims=True), 1e-12)
    acc = jnp.dot(a, x_ref[...], preferred_element_type=jnp.float32)
    h = jnp.maximum(
        jnp.dot(acc / deg, w1_ref[...], preferred_element_type=jnp.float32)
        + b1_ref[...],
        0.0,
    )
    hw2_ref[...] = jnp.dot(h, w2_ref[...], preferred_element_type=jnp.float32)
    deg_ref[...] = deg


def _pass2_body(adj_ref, hw2_ref, deg_ref, b2_ref, out_ref):
    a = adj_ref[...]
    acc = jnp.dot(a, hw2_ref[...], preferred_element_type=jnp.float32)
    out_ref[...] = acc / deg_ref[...] + b2_ref[...]


def kernel(input_matrix, adj, W1, b1, W2, b2):
    n, d = input_matrix.shape
    h_dim = W1.shape[1]
    c = W2.shape[1]
    r = 400
    grid = (n // r,)
    b1r = b1.reshape(1, h_dim)
    b2r = b2.reshape(1, c)

    hw2, deg = pl.pallas_call(
        _pass1_body,
        grid=grid,
        in_specs=[
            pl.BlockSpec((r, n), lambda i: (i, 0)),
            pl.BlockSpec((n, d), lambda i: (0, 0)),
            pl.BlockSpec((d, h_dim), lambda i: (0, 0)),
            pl.BlockSpec((1, h_dim), lambda i: (0, 0)),
            pl.BlockSpec((h_dim, c), lambda i: (0, 0)),
        ],
        out_specs=[
            pl.BlockSpec((r, c), lambda i: (i, 0)),
            pl.BlockSpec((r, 1), lambda i: (i, 0)),
        ],
        out_shape=[
            jax.ShapeDtypeStruct((n, c), jnp.float32),
            jax.ShapeDtypeStruct((n, 1), jnp.float32),
        ],
    )(adj, input_matrix, W1, b1r, W2)

    out = pl.pallas_call(
        _pass2_body,
        grid=grid,
        in_specs=[
            pl.BlockSpec((r, n), lambda i: (i, 0)),
            pl.BlockSpec((n, c), lambda i: (0, 0)),
            pl.BlockSpec((r, 1), lambda i: (i, 0)),
            pl.BlockSpec((1, c), lambda i: (0, 0)),
        ],
        out_specs=pl.BlockSpec((r, c), lambda i: (i, 0)),
        out_shape=jax.ShapeDtypeStruct((n, c), jnp.float32),
    )(adj, hw2, deg, b2r)
    return out
